# baseline (speedup 1.0000x reference)
import functools

import jax
import jax.numpy as jnp
from jax import lax
from jax.experimental import pallas as pl
from jax.experimental.pallas import tpu as pltpu
from jax.experimental.pallas import tpu_sc as plsc

_I32_MIN = -2147483648


def _monotone_key(x):
    b = lax.bitcast_convert_type(x, jnp.int32)
    return b ^ (lax.shift_right_arithmetic(b, 31) & 0x7FFFFFFF)


def _make_sc_select(b, n):
    nch0 = n // 16
    mesh = plsc.VectorSubcoreMesh(
        core_axis_name="c", subcore_axis_name="s", num_cores=2, num_subcores=16
    )

    @functools.partial(
        pl.kernel,
        out_type=jax.ShapeDtypeStruct((2 * b, 16), jnp.int32),
        mesh=mesh,
        compiler_params=pltpu.CompilerParams(needs_layout_passes=False),
        scratch_types=[
            pltpu.VMEM((n,), jnp.float32),
            pltpu.VMEM((n,), jnp.float32),
            pltpu.VMEM((n,), jnp.float32),
            pltpu.VMEM((n + 16,), jnp.int32),
            pltpu.VMEM((16,), jnp.int32),
        ],
    )
    def sc_select(sc_hbm, gt_hbm, tm_hbm, out_hbm, sc_v, gt_v, tm_v, buf, outv):
        cidx = lax.axis_index("c")
        sidx = lax.axis_index("s")
        wid = sidx * 2 + cidx
        samp = sidx
        is_b = cidx == 1

        pltpu.sync_copy(sc_hbm.at[samp], sc_v)
        pltpu.sync_copy(gt_hbm.at[samp], gt_v)
        pltpu.sync_copy(tm_hbm.at[samp], tm_v)


        isb_m = jnp.broadcast_to(is_b.astype(jnp.int32), (16,)) > 0
        ones_v = jnp.full((16,), 1, jnp.int32)
        zero_v = jnp.zeros((16,), jnp.int32)

        def pass0(j, carry):
            cp, cn = carry
            sl = pl.ds(j * 16, 16)
            g = gt_v[sl]
            t = tm_v[sl]
            key = _monotone_key(sc_v[sl])
            negm = g < 0.5
            cp = cp + jnp.where(jnp.logical_and(g > 0.5, t > 0.5), ones_v, zero_v)
            cn = cn + jnp.where(negm, ones_v, zero_v)
            key = jnp.where(
                jnp.logical_and(isb_m, jnp.logical_not(negm)), _I32_MIN, key
            )
            buf[sl] = key
            return cp, cn

        cp, cn = lax.fori_loop(
            0, nch0, pass0,
            (jnp.zeros((16,), jnp.int32), jnp.zeros((16,), jnp.int32)),
        )
        pos_num = jnp.sum(cp)
        neg_full = jnp.sum(cn)
        idx_a = lax.shift_right_arithmetic(neg_full, 1) - 1
        idx_a = jnp.where(idx_a < 0, idx_a + n, idx_a)
        k_a = idx_a + 1
        k_b = jnp.maximum(jnp.minimum(pos_num * 3, neg_full), 1)
        k = jnp.where(is_b, k_b, k_a)

        def round_body(_, carry):
            prefix, bit, nn, above = carry
            trial = prefix | bit
            thr_v = jnp.broadcast_to(trial ^ _I32_MIN, (16,))
            nch = lax.shift_right_logical(nn, 4)

            def cbody(j, cv):
                return cv + jnp.where(buf[pl.ds(j * 16, 16)] >= thr_v, ones_v, zero_v)

            c = jnp.sum(lax.fori_loop(0, nch, cbody, jnp.zeros((16,), jnp.int32)))
            take = (above + c) >= k
            take_m = jnp.broadcast_to(take.astype(jnp.int32), (16,)) > 0

            def kbody(j, woff):
                kk = buf[pl.ds(j * 16, 16)]
                m = jnp.logical_not(jnp.logical_xor(kk >= thr_v, take_m))
                plsc.store_compressed(buf.at[pl.ds(woff, 16)], kk, mask=m)
                return woff + jnp.sum(jnp.where(m, ones_v, zero_v))

            woff = lax.fori_loop(0, nch, kbody, jnp.int32(0))
            buf[pl.ds(woff, 16)] = jnp.full((16,), _I32_MIN, jnp.int32)
            nn2 = lax.shift_left(lax.shift_right_logical(woff + 15, 4), 4)
            prefix = jnp.where(take, trial, prefix)
            above = jnp.where(take, above, above + c)
            return prefix, lax.shift_right_logical(bit, 1), nn2, above

        prefix, _, _, _ = lax.fori_loop(
            0, 32, round_body,
            (jnp.int32(0), jnp.int32(_I32_MIN), jnp.int32(n), jnp.int32(0)),
        )

        outv[...] = jnp.broadcast_to(prefix ^ _I32_MIN, (16,))
        pltpu.sync_copy(outv, out_hbm.at[wid])

    return sc_select


def _tc_body(gt_ref, sc_ref, tm_ref, ytg_ref, ypg_ref, thr_ref, out_ref, acc_ref):
    i = pl.program_id(0)
    nb = pl.num_programs(0)

    gt = gt_ref[0]
    sc = sc_ref[0]
    tm = tm_ref[0]
    n = gt.shape[0] * gt.shape[1]

    thr_a = thr_ref[2 * i, 0]
    thr_b = thr_ref[2 * i + 1, 0]

    skey = _monotone_key(sc)
    pos = gt > 0.5
    tmpos = tm > 0.5
    pos_num = jnp.sum((pos & tmpos).astype(jnp.int32))
    neg_full = jnp.sum((gt < 0.5).astype(jnp.int32))
    neg_num = jnp.minimum(pos_num * 3, neg_full)

    mask_a = (skey >= thr_a).astype(jnp.float32)
    mask_b = (((skey >= thr_b) | pos) & tmpos).astype(jnp.float32)
    ohem = jnp.where(pos_num == 0, mask_a, jnp.where(neg_num == 0, tm, mask_b))

    di = jnp.sum(gt * sc * ohem)
    du1 = jnp.sum(gt * ohem)
    du2 = jnp.sum(sc * ohem)

    d1g = ytg_ref[0, 0]
    d2g = ytg_ref[0, 1]
    d3g = ytg_ref[0, 2]
    d4g = ytg_ref[0, 3]
    thg = ytg_ref[0, 4]
    d1p = ypg_ref[0, 0]
    d2p = ypg_ref[0, 1]
    d3p = ypg_ref[0, 2]
    d4p = ypg_ref[0, 3]
    thp = ypg_ref[0, 4]

    area_gt = (d1g + d3g) * (d2g + d4g)
    area_pred = (d1p + d3p) * (d2p + d4p)
    w_union = jnp.minimum(d2g, d2p) + jnp.minimum(d4g, d4p)
    h_union = jnp.minimum(d1g, d1p) + jnp.minimum(d3g, d3p)
    area_i = w_union * h_union
    area_u = area_gt + area_pred - area_i
    l_aabb = -jnp.log((area_i + 1.0) / (area_u + 1.0))
    l_theta = 1.0 - jnp.cos(thp - thg)
    l_g = l_aabb + 20.0 * l_theta
    g_part = jnp.sum(l_g * gt * tm)

    @pl.when(i == 0)
    def _init():
        acc_ref[0] = 0.0
        acc_ref[1] = 0.0
        acc_ref[2] = 0.0
        acc_ref[3] = 0.0

    acc_ref[0] = acc_ref[0] + g_part
    acc_ref[1] = acc_ref[1] + di
    acc_ref[2] = acc_ref[2] + du1
    acc_ref[3] = acc_ref[3] + du2

    @pl.when(i == nb - 1)
    def _fin():
        union = acc_ref[2] + acc_ref[3] + 1e-5
        cls = (1.0 - 2.0 * acc_ref[1] / union) * 0.01
        out_ref[0, 0] = acc_ref[0] / (nb * n) + cls


def kernel(y_true_cls, y_pred_cls, y_true_geo, y_pred_geo, training_mask):
    b, _, h, w = y_true_cls.shape
    n = h * w

    thr = _make_sc_select(b, n)(
        y_pred_cls.reshape(b, n),
        y_true_cls.reshape(b, n),
        training_mask.reshape(b, n),
    )

    out = pl.pallas_call(
        _tc_body,
        grid=(b,),
        in_specs=[
            pl.BlockSpec((1, h, w), lambda i: (i, 0, 0)),
            pl.BlockSpec((1, h, w), lambda i: (i, 0, 0)),
            pl.BlockSpec((1, h, w), lambda i: (i, 0, 0)),
            pl.BlockSpec((1, 5, h, w), lambda i: (i, 0, 0, 0)),
            pl.BlockSpec((1, 5, h, w), lambda i: (i, 0, 0, 0)),
            pl.BlockSpec(memory_space=pltpu.SMEM),
        ],
        out_specs=pl.BlockSpec(memory_space=pltpu.SMEM),
        out_shape=jax.ShapeDtypeStruct((1, 1), jnp.float32),
        scratch_shapes=[pltpu.SMEM((4,), jnp.float32)],
    )(
        y_true_cls.reshape(b, h, w),
        y_pred_cls.reshape(b, h, w),
        training_mask.reshape(b, h, w),
        y_true_geo,
        y_pred_geo,
        thr,
    )
    return out[0, 0]


# speedup vs baseline: 2.1616x; 2.1616x over previous
import functools

import jax
import jax.numpy as jnp
from jax import lax
from jax.experimental import pallas as pl
from jax.experimental.pallas import tpu as pltpu
from jax.experimental.pallas import tpu_sc as plsc

_I32_MIN = -2147483648


def _monotone_key(x):
    b = lax.bitcast_convert_type(x, jnp.int32)
    return b ^ (lax.shift_right_arithmetic(b, 31) & 0x7FFFFFFF)


def _make_sc_select(b, n):
    nch0 = n // 16
    mesh = plsc.VectorSubcoreMesh(
        core_axis_name="c", subcore_axis_name="s", num_cores=2, num_subcores=16
    )

    @functools.partial(
        pl.kernel,
        out_type=jax.ShapeDtypeStruct((2 * b, 16), jnp.int32),
        mesh=mesh,
        compiler_params=pltpu.CompilerParams(needs_layout_passes=False),
        scratch_types=[
            pltpu.VMEM((n,), jnp.float32),
            pltpu.VMEM((n,), jnp.float32),
            pltpu.VMEM((n,), jnp.float32),
            pltpu.VMEM((n,), jnp.int32),
            pltpu.VMEM((4096,), jnp.int32),
            pltpu.VMEM((256,), jnp.int32),
            pltpu.VMEM((16,), jnp.int32),
        ],
    )
    def sc_select(sc_hbm, gt_hbm, tm_hbm, out_hbm, sc_v, gt_v, tm_v, buf,
                  hist, merged, outv):
        cidx = lax.axis_index("c")
        sidx = lax.axis_index("s")
        wid = sidx * 2 + cidx
        samp = sidx
        is_b = cidx == 1

        pltpu.sync_copy(sc_hbm.at[samp], sc_v)
        pltpu.sync_copy(gt_hbm.at[samp], gt_v)
        pltpu.sync_copy(tm_hbm.at[samp], tm_v)

        isb_m = jnp.broadcast_to(is_b.astype(jnp.int32), (16,)) > 0
        ones_v = jnp.full((16,), 1, jnp.int32)
        zero_v = jnp.zeros((16,), jnp.int32)
        lane_iota = lax.iota(jnp.int32, 16)
        laneoff = lane_iota * 256

        def pass0(j, carry):
            cp, cn = carry
            sl = pl.ds(j * 16, 16)
            g = gt_v[sl]
            t = tm_v[sl]
            key = _monotone_key(sc_v[sl]) ^ _I32_MIN
            negm = g < 0.5
            cp = cp + jnp.where(jnp.logical_and(g > 0.5, t > 0.5), ones_v, zero_v)
            cn = cn + jnp.where(negm, ones_v, zero_v)
            key = jnp.where(
                jnp.logical_and(isb_m, jnp.logical_not(negm)), 0, key
            )
            buf[sl] = key
            return cp, cn

        cp, cn = lax.fori_loop(
            0, nch0, pass0,
            (jnp.zeros((16,), jnp.int32), jnp.zeros((16,), jnp.int32)),
        )
        pos_num = jnp.sum(cp)
        neg_full = jnp.sum(cn)
        idx_a = lax.shift_right_arithmetic(neg_full, 1) - 1
        idx_a = jnp.where(idx_a < 0, idx_a + n, idx_a)
        k_a = idx_a + 1
        k_b = jnp.maximum(jnp.minimum(pos_num * 3, neg_full), 1)
        k = jnp.where(is_b, k_b, k_a)

        k_rem = k
        pfx = jnp.int32(0)
        for p in range(4):
            shift = 24 - 8 * p
            pfx_v = jnp.broadcast_to(pfx, (16,))

            def zbody(i, _):
                for u in range(4):
                    hist[pl.ds((i * 4 + u) * 16, 16)] = zero_v
                return 0

            lax.fori_loop(0, 64, zbody, 0)

            def sbody(j, _, p=p, shift=shift, pfx_v=pfx_v):
                for u in range(4):
                    sl = pl.ds((j * 4 + u) * 16, 16)
                    bk = buf[sl]
                    bins = lax.shift_right_logical(bk, shift) & 255
                    addr = laneoff + bins
                    if p == 0:
                        plsc.addupdate_scatter(hist, [addr], ones_v)
                    else:
                        m = lax.shift_right_logical(bk, 32 - 8 * p) == pfx_v
                        plsc.addupdate_scatter(hist, [addr], ones_v, mask=m)
                return 0

            lax.fori_loop(0, nch0 // 4, sbody, 0)

            def mbody(c, csums):
                acc = zero_v
                for r in range(16):
                    acc = acc + hist[pl.ds(r * 256 + c * 16, 16)]
                merged[pl.ds(c * 16, 16)] = acc
                cs = jnp.sum(acc)
                onehot = lane_iota == jnp.broadcast_to(c, (16,))
                return csums + jnp.where(onehot, jnp.broadcast_to(cs, (16,)), zero_v)

            csums = lax.fori_loop(0, 16, mbody, zero_v)

            k_rem_v = jnp.broadcast_to(k_rem, (16,))

            pre_c = plsc.cumsum(csums)
            total_v = jnp.broadcast_to(jnp.sum(csums), (16,))
            above_cv = total_v - pre_c
            hit_c = jnp.logical_and(above_cv < k_rem_v,
                                    (above_cv + csums) >= k_rem_v)
            chunk_sel = jnp.sum(jnp.where(hit_c, lane_iota, zero_v))
            above_c = jnp.sum(jnp.where(hit_c, above_cv, zero_v))

            hvec = merged[pl.ds(chunk_sel * 16, 16)]
            pre_b = plsc.cumsum(hvec)
            subtot_v = jnp.broadcast_to(above_c + jnp.sum(hvec), (16,))
            above_bv = subtot_v - pre_b
            hit_b = jnp.logical_and(above_bv < k_rem_v,
                                    (above_bv + hvec) >= k_rem_v)
            bin_sel = jnp.sum(jnp.where(hit_b, lane_iota, zero_v))
            above_t = jnp.sum(jnp.where(hit_b, above_bv, zero_v))

            pfx = lax.shift_left(pfx, 8) | (chunk_sel * 16 + bin_sel)
            k_rem = k_rem - above_t

        outv[...] = jnp.broadcast_to(pfx ^ _I32_MIN, (16,))
        pltpu.sync_copy(outv, out_hbm.at[wid])

    return sc_select


def _tc_body(gt_ref, sc_ref, tm_ref, ytg_ref, ypg_ref, thr_ref, out_ref, acc_ref):
    i = pl.program_id(0)
    nb = pl.num_programs(0)

    gt = gt_ref[0]
    sc = sc_ref[0]
    tm = tm_ref[0]
    n = gt.shape[0] * gt.shape[1]

    thr_a = thr_ref[2 * i, 0]
    thr_b = thr_ref[2 * i + 1, 0]

    skey = _monotone_key(sc)
    pos = gt > 0.5
    tmpos = tm > 0.5
    pos_num = jnp.sum((pos & tmpos).astype(jnp.int32))
    neg_full = jnp.sum((gt < 0.5).astype(jnp.int32))
    neg_num = jnp.minimum(pos_num * 3, neg_full)

    mask_a = (skey >= thr_a).astype(jnp.float32)
    mask_b = (((skey >= thr_b) | pos) & tmpos).astype(jnp.float32)
    ohem = jnp.where(pos_num == 0, mask_a, jnp.where(neg_num == 0, tm, mask_b))

    di = jnp.sum(gt * sc * ohem)
    du1 = jnp.sum(gt * ohem)
    du2 = jnp.sum(sc * ohem)

    d1g = ytg_ref[0, 0]
    d2g = ytg_ref[0, 1]
    d3g = ytg_ref[0, 2]
    d4g = ytg_ref[0, 3]
    thg = ytg_ref[0, 4]
    d1p = ypg_ref[0, 0]
    d2p = ypg_ref[0, 1]
    d3p = ypg_ref[0, 2]
    d4p = ypg_ref[0, 3]
    thp = ypg_ref[0, 4]

    area_gt = (d1g + d3g) * (d2g + d4g)
    area_pred = (d1p + d3p) * (d2p + d4p)
    w_union = jnp.minimum(d2g, d2p) + jnp.minimum(d4g, d4p)
    h_union = jnp.minimum(d1g, d1p) + jnp.minimum(d3g, d3p)
    area_i = w_union * h_union
    area_u = area_gt + area_pred - area_i
    l_aabb = -jnp.log((area_i + 1.0) / (area_u + 1.0))
    l_theta = 1.0 - jnp.cos(thp - thg)
    l_g = l_aabb + 20.0 * l_theta
    g_part = jnp.sum(l_g * gt * tm)

    @pl.when(i == 0)
    def _init():
        acc_ref[0] = 0.0
        acc_ref[1] = 0.0
        acc_ref[2] = 0.0
        acc_ref[3] = 0.0

    acc_ref[0] = acc_ref[0] + g_part
    acc_ref[1] = acc_ref[1] + di
    acc_ref[2] = acc_ref[2] + du1
    acc_ref[3] = acc_ref[3] + du2

    @pl.when(i == nb - 1)
    def _fin():
        union = acc_ref[2] + acc_ref[3] + 1e-5
        cls = (1.0 - 2.0 * acc_ref[1] / union) * 0.01
        out_ref[0, 0] = acc_ref[0] / (nb * n) + cls


def kernel(y_true_cls, y_pred_cls, y_true_geo, y_pred_geo, training_mask):
    b, _, h, w = y_true_cls.shape
    n = h * w

    thr = _make_sc_select(b, n)(
        y_pred_cls.reshape(b, n),
        y_true_cls.reshape(b, n),
        training_mask.reshape(b, n),
    )

    out = pl.pallas_call(
        _tc_body,
        grid=(b,),
        in_specs=[
            pl.BlockSpec((1, h, w), lambda i: (i, 0, 0)),
            pl.BlockSpec((1, h, w), lambda i: (i, 0, 0)),
            pl.BlockSpec((1, h, w), lambda i: (i, 0, 0)),
            pl.BlockSpec((1, 5, h, w), lambda i: (i, 0, 0, 0)),
            pl.BlockSpec((1, 5, h, w), lambda i: (i, 0, 0, 0)),
            pl.BlockSpec(memory_space=pltpu.SMEM),
        ],
        out_specs=pl.BlockSpec(memory_space=pltpu.SMEM),
        out_shape=jax.ShapeDtypeStruct((1, 1), jnp.float32),
        scratch_shapes=[pltpu.SMEM((4,), jnp.float32)],
    )(
        y_true_cls.reshape(b, h, w),
        y_pred_cls.reshape(b, h, w),
        training_mask.reshape(b, h, w),
        y_true_geo,
        y_pred_geo,
        thr,
    )
    return out[0, 0]


# speedup vs baseline: 2.2980x; 1.0631x over previous
import functools

import jax
import jax.numpy as jnp
from jax import lax
from jax.experimental import pallas as pl
from jax.experimental.pallas import tpu as pltpu
from jax.experimental.pallas import tpu_sc as plsc

_I32_MIN = -2147483648


def _monotone_key(x):
    b = lax.bitcast_convert_type(x, jnp.int32)
    return b ^ (lax.shift_right_arithmetic(b, 31) & 0x7FFFFFFF)


def _make_sc_select(b, n):
    nch0 = n // 16
    mesh = plsc.VectorSubcoreMesh(
        core_axis_name="c", subcore_axis_name="s", num_cores=2, num_subcores=16
    )

    @functools.partial(
        pl.kernel,
        out_type=jax.ShapeDtypeStruct((2 * b, 16), jnp.int32),
        mesh=mesh,
        compiler_params=pltpu.CompilerParams(needs_layout_passes=False),
        scratch_types=[
            pltpu.VMEM((n,), jnp.float32),
            pltpu.VMEM((n,), jnp.float32),
            pltpu.VMEM((n,), jnp.float32),
            pltpu.VMEM((n,), jnp.int32),
            pltpu.VMEM((4096,), jnp.int32),
            pltpu.VMEM((256,), jnp.int32),
            pltpu.VMEM((16,), jnp.int32),
        ],
    )
    def sc_select(sc_hbm, gt_hbm, tm_hbm, out_hbm, sc_v, gt_v, tm_v, buf,
                  hist, merged, outv):
        cidx = lax.axis_index("c")
        sidx = lax.axis_index("s")
        wid = sidx * 2 + cidx
        samp = sidx
        is_b = cidx == 1

        pltpu.sync_copy(sc_hbm.at[samp], sc_v)
        pltpu.sync_copy(gt_hbm.at[samp], gt_v)
        pltpu.sync_copy(tm_hbm.at[samp], tm_v)

        isb_m = jnp.broadcast_to(is_b.astype(jnp.int32), (16,)) > 0
        ones_v = jnp.full((16,), 1, jnp.int32)
        zero_v = jnp.zeros((16,), jnp.int32)
        lane_iota = lax.iota(jnp.int32, 16)
        laneoff = lane_iota * 256

        def pass0(j, carry):
            cp, cn = carry
            for u in range(4):
                sl = pl.ds((j * 4 + u) * 16, 16)
                g = gt_v[sl]
                t = tm_v[sl]
                key = _monotone_key(sc_v[sl]) ^ _I32_MIN
                negm = g < 0.5
                cp = cp + jnp.where(jnp.logical_and(g > 0.5, t > 0.5), ones_v, zero_v)
                cn = cn + jnp.where(negm, ones_v, zero_v)
                key = jnp.where(
                    jnp.logical_and(isb_m, jnp.logical_not(negm)), 0, key
                )
                buf[sl] = key
            return cp, cn

        cp, cn = lax.fori_loop(
            0, nch0 // 4, pass0,
            (jnp.zeros((16,), jnp.int32), jnp.zeros((16,), jnp.int32)),
        )
        pos_num = jnp.sum(cp)
        neg_full = jnp.sum(cn)
        idx_a = lax.shift_right_arithmetic(neg_full, 1) - 1
        idx_a = jnp.where(idx_a < 0, idx_a + n, idx_a)
        k_a = idx_a + 1
        k_b = jnp.maximum(jnp.minimum(pos_num * 3, neg_full), 1)
        k = jnp.where(is_b, k_b, k_a)

        k_rem = k
        pfx = jnp.int32(0)
        for p in range(4):
            shift = 24 - 8 * p
            pfx_v = jnp.broadcast_to(pfx, (16,))

            def zbody(i, _):
                for u in range(4):
                    hist[pl.ds((i * 4 + u) * 16, 16)] = zero_v
                return 0

            lax.fori_loop(0, 64, zbody, 0)

            def sbody(j, _, p=p, shift=shift, pfx_v=pfx_v):
                for u in range(8):
                    sl = pl.ds((j * 8 + u) * 16, 16)
                    bk = buf[sl]
                    bins = lax.shift_right_logical(bk, shift) & 255
                    addr = laneoff + bins
                    if p == 0:
                        plsc.addupdate_scatter(hist, [addr], ones_v)
                    else:
                        m = lax.shift_right_logical(bk, 32 - 8 * p) == pfx_v
                        plsc.addupdate_scatter(hist, [addr], ones_v, mask=m)
                return 0

            lax.fori_loop(0, nch0 // 8, sbody, 0)

            def mbody(c, csums):
                acc = zero_v
                for r in range(16):
                    acc = acc + hist[pl.ds(r * 256 + c * 16, 16)]
                merged[pl.ds(c * 16, 16)] = acc
                cs = jnp.sum(acc)
                onehot = lane_iota == jnp.broadcast_to(c, (16,))
                return csums + jnp.where(onehot, jnp.broadcast_to(cs, (16,)), zero_v)

            csums = lax.fori_loop(0, 16, mbody, zero_v)

            k_rem_v = jnp.broadcast_to(k_rem, (16,))

            pre_c = plsc.cumsum(csums)
            total_v = jnp.broadcast_to(jnp.sum(csums), (16,))
            above_cv = total_v - pre_c
            hit_c = jnp.logical_and(above_cv < k_rem_v,
                                    (above_cv + csums) >= k_rem_v)
            chunk_sel = jnp.sum(jnp.where(hit_c, lane_iota, zero_v))
            above_c = jnp.sum(jnp.where(hit_c, above_cv, zero_v))

            hvec = merged[pl.ds(chunk_sel * 16, 16)]
            pre_b = plsc.cumsum(hvec)
            subtot_v = jnp.broadcast_to(above_c + jnp.sum(hvec), (16,))
            above_bv = subtot_v - pre_b
            hit_b = jnp.logical_and(above_bv < k_rem_v,
                                    (above_bv + hvec) >= k_rem_v)
            bin_sel = jnp.sum(jnp.where(hit_b, lane_iota, zero_v))
            above_t = jnp.sum(jnp.where(hit_b, above_bv, zero_v))

            pfx = lax.shift_left(pfx, 8) | (chunk_sel * 16 + bin_sel)
            k_rem = k_rem - above_t

        out_vec = jnp.where(
            lane_iota == 1,
            jnp.broadcast_to(pos_num, (16,)),
            jnp.broadcast_to(pfx ^ _I32_MIN, (16,)),
        )
        out_vec = jnp.where(
            lane_iota == 2, jnp.broadcast_to(neg_full, (16,)), out_vec
        )
        outv[...] = out_vec
        pltpu.sync_copy(outv, out_hbm.at[wid])

    return sc_select


def _tc_dice_body(gt_ref, sc_ref, tm_ref, thr_ref, g_ref, out_ref, acc_ref):
    i = pl.program_id(0)
    nb = pl.num_programs(0)

    gt = gt_ref[0]
    sc = sc_ref[0]
    tm = tm_ref[0]
    n = gt.shape[0] * gt.shape[1]

    thr_a = thr_ref[2 * i, 0]
    thr_b = thr_ref[2 * i + 1, 0]
    pos_num = thr_ref[2 * i, 1]
    neg_full = thr_ref[2 * i, 2]
    neg_num = jnp.minimum(pos_num * 3, neg_full)

    skey = _monotone_key(sc)
    pos = gt > 0.5
    tmpos = tm > 0.5

    mask_a = (skey >= thr_a).astype(jnp.float32)
    mask_b = (((skey >= thr_b) | pos) & tmpos).astype(jnp.float32)
    ohem = jnp.where(pos_num == 0, mask_a, jnp.where(neg_num == 0, tm, mask_b))

    di = jnp.sum(gt * sc * ohem)
    du1 = jnp.sum(gt * ohem)
    du2 = jnp.sum(sc * ohem)

    @pl.when(i == 0)
    def _init():
        acc_ref[0] = 0.0
        acc_ref[1] = 0.0
        acc_ref[2] = 0.0

    acc_ref[0] = acc_ref[0] + di
    acc_ref[1] = acc_ref[1] + du1
    acc_ref[2] = acc_ref[2] + du2

    @pl.when(i == nb - 1)
    def _fin():
        union = acc_ref[1] + acc_ref[2] + 1e-5
        cls = (1.0 - 2.0 * acc_ref[0] / union) * 0.01
        out_ref[0, 0] = g_ref[0, 0] / (nb * n) + cls


def _tc_geo_body(gt_ref, tm_ref, ytg_ref, ypg_ref, out_ref, acc_ref):
    i = pl.program_id(0)
    nb = pl.num_programs(0)

    gt = gt_ref[0]
    tm = tm_ref[0]

    d1g = ytg_ref[0, 0]
    d2g = ytg_ref[0, 1]
    d3g = ytg_ref[0, 2]
    d4g = ytg_ref[0, 3]
    thg = ytg_ref[0, 4]
    d1p = ypg_ref[0, 0]
    d2p = ypg_ref[0, 1]
    d3p = ypg_ref[0, 2]
    d4p = ypg_ref[0, 3]
    thp = ypg_ref[0, 4]

    area_gt = (d1g + d3g) * (d2g + d4g)
    area_pred = (d1p + d3p) * (d2p + d4p)
    w_union = jnp.minimum(d2g, d2p) + jnp.minimum(d4g, d4p)
    h_union = jnp.minimum(d1g, d1p) + jnp.minimum(d3g, d3p)
    area_i = w_union * h_union
    area_u = area_gt + area_pred - area_i
    l_aabb = -jnp.log((area_i + 1.0) / (area_u + 1.0))
    l_theta = 1.0 - jnp.cos(thp - thg)
    l_g = l_aabb + 20.0 * l_theta
    g_part = jnp.sum(l_g * gt * tm)

    @pl.when(i == 0)
    def _init():
        acc_ref[0] = 0.0

    acc_ref[0] = acc_ref[0] + g_part

    @pl.when(i == nb - 1)
    def _fin():
        out_ref[0, 0] = acc_ref[0]


def kernel(y_true_cls, y_pred_cls, y_true_geo, y_pred_geo, training_mask):
    b, _, h, w = y_true_cls.shape
    n = h * w

    gt3 = y_true_cls.reshape(b, h, w)
    sc3 = y_pred_cls.reshape(b, h, w)
    tm3 = training_mask.reshape(b, h, w)

    thr = _make_sc_select(b, n)(
        y_pred_cls.reshape(b, n),
        y_true_cls.reshape(b, n),
        training_mask.reshape(b, n),
    )

    g_total = pl.pallas_call(
        _tc_geo_body,
        grid=(b,),
        in_specs=[
            pl.BlockSpec((1, h, w), lambda i: (i, 0, 0)),
            pl.BlockSpec((1, h, w), lambda i: (i, 0, 0)),
            pl.BlockSpec((1, 5, h, w), lambda i: (i, 0, 0, 0)),
            pl.BlockSpec((1, 5, h, w), lambda i: (i, 0, 0, 0)),
        ],
        out_specs=pl.BlockSpec(memory_space=pltpu.SMEM),
        out_shape=jax.ShapeDtypeStruct((1, 1), jnp.float32),
        scratch_shapes=[pltpu.SMEM((1,), jnp.float32)],
    )(gt3, tm3, y_true_geo, y_pred_geo)

    out = pl.pallas_call(
        _tc_dice_body,
        grid=(b,),
        in_specs=[
            pl.BlockSpec((1, h, w), lambda i: (i, 0, 0)),
            pl.BlockSpec((1, h, w), lambda i: (i, 0, 0)),
            pl.BlockSpec((1, h, w), lambda i: (i, 0, 0)),
            pl.BlockSpec(memory_space=pltpu.SMEM),
            pl.BlockSpec(memory_space=pltpu.SMEM),
        ],
        out_specs=pl.BlockSpec(memory_space=pltpu.SMEM),
        out_shape=jax.ShapeDtypeStruct((1, 1), jnp.float32),
        scratch_shapes=[pltpu.SMEM((4,), jnp.float32)],
    )(gt3, sc3, tm3, thr, g_total)
    return out[0, 0]


# speedup vs baseline: 2.4144x; 1.0507x over previous
import functools

import jax
import jax.numpy as jnp
from jax import lax
from jax.experimental import pallas as pl
from jax.experimental.pallas import tpu as pltpu
from jax.experimental.pallas import tpu_sc as plsc

_I32_MIN = -2147483648


def _monotone_key(x):
    b = lax.bitcast_convert_type(x, jnp.int32)
    return b ^ (lax.shift_right_arithmetic(b, 31) & 0x7FFFFFFF)


def _make_sc_select(b, n):
    nch0 = n // 16
    mesh = plsc.VectorSubcoreMesh(
        core_axis_name="c", subcore_axis_name="s", num_cores=2, num_subcores=16
    )

    @functools.partial(
        pl.kernel,
        out_type=jax.ShapeDtypeStruct((2 * b, 16), jnp.int32),
        mesh=mesh,
        compiler_params=pltpu.CompilerParams(needs_layout_passes=False),
        scratch_types=[
            pltpu.VMEM((n,), jnp.float32),
            pltpu.VMEM((n,), jnp.float32),
            pltpu.VMEM((n,), jnp.float32),
            pltpu.VMEM((n,), jnp.int32),
            pltpu.VMEM((4096,), jnp.int32),
            pltpu.VMEM((16,), jnp.int32),
        ],
    )
    def sc_select(sc_hbm, gt_hbm, tm_hbm, out_hbm, sc_v, gt_v, tm_v, buf,
                  hist, outv):
        cidx = lax.axis_index("c")
        sidx = lax.axis_index("s")
        wid = sidx * 2 + cidx
        samp = sidx
        is_b = cidx == 1

        pltpu.sync_copy(sc_hbm.at[samp], sc_v)
        pltpu.sync_copy(gt_hbm.at[samp], gt_v)
        pltpu.sync_copy(tm_hbm.at[samp], tm_v)

        isb_m = jnp.broadcast_to(is_b.astype(jnp.int32), (16,)) > 0
        ones_v = jnp.full((16,), 1, jnp.int32)
        zero_v = jnp.zeros((16,), jnp.int32)
        lane_iota = lax.iota(jnp.int32, 16)

        def pass0(j, carry):
            cp, cn = carry
            for u in range(4):
                sl = pl.ds((j * 4 + u) * 16, 16)
                g = gt_v[sl]
                t = tm_v[sl]
                key = _monotone_key(sc_v[sl]) ^ _I32_MIN
                negm = g < 0.5
                cp = cp + jnp.where(jnp.logical_and(g > 0.5, t > 0.5), ones_v, zero_v)
                cn = cn + jnp.where(negm, ones_v, zero_v)
                key = jnp.where(
                    jnp.logical_and(isb_m, jnp.logical_not(negm)), 0, key
                )
                buf[sl] = key
            return cp, cn

        cp, cn = lax.fori_loop(
            0, nch0 // 4, pass0,
            (jnp.zeros((16,), jnp.int32), jnp.zeros((16,), jnp.int32)),
        )
        pos_num = jnp.sum(cp)
        neg_full = jnp.sum(cn)
        idx_a = lax.shift_right_arithmetic(neg_full, 1) - 1
        idx_a = jnp.where(idx_a < 0, idx_a + n, idx_a)
        k_a = idx_a + 1
        k_b = jnp.maximum(jnp.minimum(pos_num * 3, neg_full), 1)
        k = jnp.where(is_b, k_b, k_a)

        k_rem = k
        pfx = jnp.int32(0)
        for p in range(4):
            shift = 24 - 8 * p
            pfx_v = jnp.broadcast_to(pfx, (16,))

            def zbody(i, _):
                for u in range(4):
                    hist[pl.ds((i * 4 + u) * 16, 16)] = zero_v
                return 0

            lax.fori_loop(0, 64, zbody, 0)

            def sbody(j, _, p=p, shift=shift, pfx_v=pfx_v):
                for u in range(8):
                    sl = pl.ds((j * 8 + u) * 16, 16)
                    bk = buf[sl]
                    bins = lax.shift_right_logical(bk, shift) & 255
                    addr = lax.shift_left(bins, 4) | lane_iota
                    if p == 0:
                        plsc.addupdate_scatter(hist, [addr], ones_v)
                    else:
                        m = lax.shift_right_logical(bk, 32 - 8 * p) == pfx_v
                        plsc.addupdate_scatter(hist, [addr], ones_v, mask=m)
                return 0

            lax.fori_loop(0, nch0 // 8, sbody, 0)

            def mbody(c, csums):
                acc = zero_v
                for j in range(16):
                    acc = acc + hist[pl.ds((c * 16 + j) * 16, 16)]
                cs = jnp.sum(acc)
                onehot = lane_iota == jnp.broadcast_to(c, (16,))
                return csums + jnp.where(onehot, jnp.broadcast_to(cs, (16,)), zero_v)

            csums = lax.fori_loop(0, 16, mbody, zero_v)

            k_rem_v = jnp.broadcast_to(k_rem, (16,))

            pre_c = plsc.cumsum(csums)
            total_v = jnp.broadcast_to(jnp.sum(csums), (16,))
            above_cv = total_v - pre_c
            hit_c = jnp.logical_and(above_cv < k_rem_v,
                                    (above_cv + csums) >= k_rem_v)
            chunk_sel = jnp.sum(jnp.where(hit_c, lane_iota, zero_v))
            above_c = jnp.sum(jnp.where(hit_c, above_cv, zero_v))

            def bbody(j, hvec, chunk_sel=chunk_sel):
                hb = jnp.sum(hist[pl.ds((chunk_sel * 16 + j) * 16, 16)])
                onehot = lane_iota == jnp.broadcast_to(j, (16,))
                return hvec + jnp.where(onehot, jnp.broadcast_to(hb, (16,)), zero_v)

            hvec = lax.fori_loop(0, 16, bbody, zero_v)
            pre_b = plsc.cumsum(hvec)
            subtot_v = jnp.broadcast_to(above_c + jnp.sum(hvec), (16,))
            above_bv = subtot_v - pre_b
            hit_b = jnp.logical_and(above_bv < k_rem_v,
                                    (above_bv + hvec) >= k_rem_v)
            bin_sel = jnp.sum(jnp.where(hit_b, lane_iota, zero_v))
            above_t = jnp.sum(jnp.where(hit_b, above_bv, zero_v))

            pfx = lax.shift_left(pfx, 8) | (chunk_sel * 16 + bin_sel)
            k_rem = k_rem - above_t

        out_vec = jnp.where(
            lane_iota == 1,
            jnp.broadcast_to(pos_num, (16,)),
            jnp.broadcast_to(pfx ^ _I32_MIN, (16,)),
        )
        out_vec = jnp.where(
            lane_iota == 2, jnp.broadcast_to(neg_full, (16,)), out_vec
        )
        outv[...] = out_vec
        pltpu.sync_copy(outv, out_hbm.at[wid])

    return sc_select


def _tc_dice_body(gt_ref, sc_ref, tm_ref, thr_ref, g_ref, out_ref, acc_ref):
    i = pl.program_id(0)
    nb = pl.num_programs(0)

    gt = gt_ref[0]
    sc = sc_ref[0]
    tm = tm_ref[0]
    n = gt.shape[0] * gt.shape[1]

    thr_a = thr_ref[2 * i, 0]
    thr_b = thr_ref[2 * i + 1, 0]
    pos_num = thr_ref[2 * i, 1]
    neg_full = thr_ref[2 * i, 2]
    neg_num = jnp.minimum(pos_num * 3, neg_full)

    skey = _monotone_key(sc)
    pos = gt > 0.5
    tmpos = tm > 0.5

    mask_a = (skey >= thr_a).astype(jnp.float32)
    mask_b = (((skey >= thr_b) | pos) & tmpos).astype(jnp.float32)
    ohem = jnp.where(pos_num == 0, mask_a, jnp.where(neg_num == 0, tm, mask_b))

    di = jnp.sum(gt * sc * ohem)
    du1 = jnp.sum(gt * ohem)
    du2 = jnp.sum(sc * ohem)

    @pl.when(i == 0)
    def _init():
        acc_ref[0] = 0.0
        acc_ref[1] = 0.0
        acc_ref[2] = 0.0

    acc_ref[0] = acc_ref[0] + di
    acc_ref[1] = acc_ref[1] + du1
    acc_ref[2] = acc_ref[2] + du2

    @pl.when(i == nb - 1)
    def _fin():
        union = acc_ref[1] + acc_ref[2] + 1e-5
        cls = (1.0 - 2.0 * acc_ref[0] / union) * 0.01
        out_ref[0, 0] = g_ref[0, 0] / (nb * n) + cls


def _tc_geo_body(gt_ref, tm_ref, ytg_ref, ypg_ref, out_ref, acc_ref):
    i = pl.program_id(0)
    nb = pl.num_programs(0)

    gt = gt_ref[0]
    tm = tm_ref[0]

    d1g = ytg_ref[0, 0]
    d2g = ytg_ref[0, 1]
    d3g = ytg_ref[0, 2]
    d4g = ytg_ref[0, 3]
    thg = ytg_ref[0, 4]
    d1p = ypg_ref[0, 0]
    d2p = ypg_ref[0, 1]
    d3p = ypg_ref[0, 2]
    d4p = ypg_ref[0, 3]
    thp = ypg_ref[0, 4]

    area_gt = (d1g + d3g) * (d2g + d4g)
    area_pred = (d1p + d3p) * (d2p + d4p)
    w_union = jnp.minimum(d2g, d2p) + jnp.minimum(d4g, d4p)
    h_union = jnp.minimum(d1g, d1p) + jnp.minimum(d3g, d3p)
    area_i = w_union * h_union
    area_u = area_gt + area_pred - area_i
    l_aabb = -jnp.log((area_i + 1.0) / (area_u + 1.0))
    l_theta = 1.0 - jnp.cos(thp - thg)
    l_g = l_aabb + 20.0 * l_theta
    g_part = jnp.sum(l_g * gt * tm)

    @pl.when(i == 0)
    def _init():
        acc_ref[0] = 0.0

    acc_ref[0] = acc_ref[0] + g_part

    @pl.when(i == nb - 1)
    def _fin():
        out_ref[0, 0] = acc_ref[0]


def kernel(y_true_cls, y_pred_cls, y_true_geo, y_pred_geo, training_mask):
    b, _, h, w = y_true_cls.shape
    n = h * w

    gt3 = y_true_cls.reshape(b, h, w)
    sc3 = y_pred_cls.reshape(b, h, w)
    tm3 = training_mask.reshape(b, h, w)

    thr = _make_sc_select(b, n)(
        y_pred_cls.reshape(b, n),
        y_true_cls.reshape(b, n),
        training_mask.reshape(b, n),
    )

    g_total = pl.pallas_call(
        _tc_geo_body,
        grid=(b,),
        in_specs=[
            pl.BlockSpec((1, h, w), lambda i: (i, 0, 0)),
            pl.BlockSpec((1, h, w), lambda i: (i, 0, 0)),
            pl.BlockSpec((1, 5, h, w), lambda i: (i, 0, 0, 0)),
            pl.BlockSpec((1, 5, h, w), lambda i: (i, 0, 0, 0)),
        ],
        out_specs=pl.BlockSpec(memory_space=pltpu.SMEM),
        out_shape=jax.ShapeDtypeStruct((1, 1), jnp.float32),
        scratch_shapes=[pltpu.SMEM((1,), jnp.float32)],
    )(gt3, tm3, y_true_geo, y_pred_geo)

    out = pl.pallas_call(
        _tc_dice_body,
        grid=(b,),
        in_specs=[
            pl.BlockSpec((1, h, w), lambda i: (i, 0, 0)),
            pl.BlockSpec((1, h, w), lambda i: (i, 0, 0)),
            pl.BlockSpec((1, h, w), lambda i: (i, 0, 0)),
            pl.BlockSpec(memory_space=pltpu.SMEM),
            pl.BlockSpec(memory_space=pltpu.SMEM),
        ],
        out_specs=pl.BlockSpec(memory_space=pltpu.SMEM),
        out_shape=jax.ShapeDtypeStruct((1, 1), jnp.float32),
        scratch_shapes=[pltpu.SMEM((4,), jnp.float32)],
    )(gt3, sc3, tm3, thr, g_total)
    return out[0, 0]
